# baseline (speedup 1.0000x reference)
import functools

import jax
import jax.numpy as jnp
from jax import lax
from jax.experimental import pallas as pl
from jax.experimental.pallas import tpu as pltpu
from jax.experimental.pallas import tpu_sc as plsc

_NC = 2
_NS = 16
_NW = _NC * _NS
_L = 16
_G = 4


@functools.lru_cache(maxsize=None)
def _make_moments_kernel(batch: int, row: int, chunk: int):
    assert batch % _NW == 0
    rpw = batch // _NW
    assert row % chunk == 0
    cpr = row // chunk
    nch = rpw * cpr
    assert chunk % (_L * _G) == 0
    iters = chunk // (_L * _G)

    mesh = plsc.VectorSubcoreMesh(core_axis_name="c", subcore_axis_name="s")

    @functools.partial(
        pl.kernel,
        out_type=jax.ShapeDtypeStruct((batch * 4 * _L,), jnp.float32),
        mesh=mesh,
        scratch_types=[
            pltpu.VMEM((chunk,), jnp.float32),
            pltpu.VMEM((chunk,), jnp.float32),
            pltpu.VMEM((rpw * 4 * _L,), jnp.float32),
            pltpu.SemaphoreType.DMA,
            pltpu.SemaphoreType.DMA,
        ],
    )
    def moments_kernel(u_hbm, out_hbm, buf0, buf1, stat_v, sem0, sem1):
        cid = lax.axis_index("c")
        sid = lax.axis_index("s")
        wid = sid * _NC + cid
        base = wid * (rpw * row)
        bufs = (buf0, buf1)
        sems = (sem0, sem1)

        def make_copy(g):
            return pltpu.make_async_copy(
                u_hbm.at[pl.ds(base + g * chunk, chunk)], bufs[g % 2],
                sems[g % 2])

        copies = [make_copy(g) for g in range(nch)]
        copies[0].start()

        for r in range(rpw):
            zero = jnp.zeros((_L,), jnp.float32)
            ss = [zero] * _G
            qq = [zero] * _G
            mn = [jnp.full((_L,), jnp.inf, jnp.float32)] * _G
            mx = [jnp.full((_L,), -jnp.inf, jnp.float32)] * _G

            for ci in range(cpr):
                g = r * cpr + ci
                if g + 1 < nch:
                    copies[g + 1].start()
                copies[g].wait()
                buf = bufs[g % 2]

                def body(i, carry, buf=buf):
                    s_c, q_c, mn_c, mx_c = carry
                    s_n, q_n, mn_n, mx_n = [], [], [], []
                    off = i * (_L * _G)
                    for k in range(_G):
                        v = buf[pl.ds(off + k * _L, _L)]
                        s_n.append(s_c[k] + v)
                        q_n.append(q_c[k] + v * v)
                        mn_n.append(jnp.minimum(mn_c[k], v))
                        mx_n.append(jnp.maximum(mx_c[k], v))
                    return (tuple(s_n), tuple(q_n), tuple(mn_n), tuple(mx_n))

                carry = lax.fori_loop(
                    0, iters, body,
                    (tuple(ss), tuple(qq), tuple(mn), tuple(mx)))
                ss, qq, mn, mx = [list(t) for t in carry]

            stat_v[pl.ds(r * 4 * _L + 0 * _L, _L)] = (
                (ss[0] + ss[1]) + (ss[2] + ss[3]))
            stat_v[pl.ds(r * 4 * _L + 1 * _L, _L)] = (
                (qq[0] + qq[1]) + (qq[2] + qq[3]))
            stat_v[pl.ds(r * 4 * _L + 2 * _L, _L)] = jnp.minimum(
                jnp.minimum(mn[0], mn[1]), jnp.minimum(mn[2], mn[3]))
            stat_v[pl.ds(r * 4 * _L + 3 * _L, _L)] = jnp.maximum(
                jnp.maximum(mx[0], mx[1]), jnp.maximum(mx[2], mx[3]))

        pltpu.sync_copy(
            stat_v, out_hbm.at[pl.ds(wid * (rpw * 4 * _L), rpw * 4 * _L)])

    return moments_kernel


def _finish_kernel(n_elems, m_ref, w1t_ref, b1_ref, w2t_ref, b2_ref, o_ref):
    m = m_ref[...]
    n = jnp.float32(n_elems)
    s = jnp.sum(m[:, 0:16], axis=1, keepdims=True)
    q = jnp.sum(m[:, 16:32], axis=1, keepdims=True)
    amin = jnp.min(m[:, 32:48], axis=1, keepdims=True)
    amax = jnp.max(m[:, 48:64], axis=1, keepdims=True)
    mean = s / n
    var = (q - s * s / n) / (n - 1.0)
    std = jnp.sqrt(jnp.maximum(var, 0.0))

    w1t = w1t_ref[...]
    h = (mean * w1t[0:1, :] + std * w1t[1:2, :]
         + amin * w1t[2:3, :] + amax * w1t[3:4, :] + b1_ref[...])
    h = jnp.maximum(h, 0.0)
    logits = jnp.dot(h, w2t_ref[...],
                     preferred_element_type=jnp.float32) + b2_ref[...]

    bsz, k = logits.shape
    col = lax.broadcasted_iota(jnp.int32, (bsz, k), 1)
    m1 = jnp.max(logits, axis=1, keepdims=True)
    i1 = jnp.min(jnp.where(logits == m1, col, k), axis=1, keepdims=True)
    l2 = jnp.where(col == i1, -jnp.inf, logits)
    m2 = jnp.max(l2, axis=1, keepdims=True)
    i2 = jnp.min(jnp.where(l2 == m2, col, k), axis=1, keepdims=True)
    e2 = jnp.exp(m2 - m1)
    denom = 1.0 + e2
    p1 = 1.0 / denom
    p2 = e2 / denom
    o_ref[...] = jnp.where(col == i1, p1,
                           jnp.where(col == i2, p2, jnp.float32(0.0)))


def kernel(u_state, W1, b1, W2, b2):
    batch = u_state.shape[0]
    row = u_state.shape[1] * u_state.shape[2]
    chunk = 32768
    u_flat = u_state.reshape(-1)

    moments = _make_moments_kernel(batch, row, chunk)(u_flat)
    moments = moments.reshape(batch, 64)

    num_prims = W2.shape[0]
    out = pl.pallas_call(
        functools.partial(_finish_kernel, row),
        out_shape=jax.ShapeDtypeStruct((batch, num_prims), jnp.float32),
    )(moments, W1.T, b1.reshape(1, -1), W2.T, b2.reshape(1, -1))
    return out


# speedup vs baseline: 81.5414x; 81.5414x over previous
import functools

import jax
import jax.numpy as jnp
from jax import lax
from jax.experimental import pallas as pl
from jax.experimental.pallas import tpu as pltpu
from jax.experimental.pallas import tpu_sc as plsc

_NC = 2
_NS = 16
_NW = _NC * _NS
_L = 16
_G = 4


@functools.lru_cache(maxsize=None)
def _make_moments_kernel(batch: int, row: int, chunk: int):
    assert batch % _NW == 0
    rpw = batch // _NW
    assert row % chunk == 0
    cpr = row // chunk
    nch = rpw * cpr
    assert chunk % (_L * _G) == 0
    iters = chunk // (_L * _G)

    mesh = plsc.VectorSubcoreMesh(core_axis_name="c", subcore_axis_name="s")

    @functools.partial(
        pl.kernel,
        out_type=jax.ShapeDtypeStruct((batch * 4 * _L,), jnp.float32),
        mesh=mesh,
        scratch_types=[
            pltpu.VMEM((chunk,), jnp.float32),
            pltpu.VMEM((chunk,), jnp.float32),
            pltpu.VMEM((rpw * 4 * _L,), jnp.float32),
            pltpu.SemaphoreType.DMA,
            pltpu.SemaphoreType.DMA,
        ],
    )
    def moments_kernel(u_hbm, out_hbm, buf0, buf1, stat_v, sem0, sem1):
        cid = lax.axis_index("c")
        sid = lax.axis_index("s")
        wid = sid * _NC + cid
        base = wid * (rpw * row)
        bufs = (buf0, buf1)
        sems = (sem0, sem1)

        def make_copy(g):
            return pltpu.make_async_copy(
                u_hbm.at[pl.ds(base + g * chunk, chunk)], bufs[g % 2],
                sems[g % 2])

        copies = [make_copy(g) for g in range(nch)]
        copies[0].start()

        for r in range(rpw):
            zero = jnp.zeros((_L,), jnp.float32)
            ss = [zero] * _G
            qq = [zero] * _G
            mn = [jnp.full((_L,), jnp.inf, jnp.float32)] * _G
            mx = [jnp.full((_L,), -jnp.inf, jnp.float32)] * _G

            for ci in range(cpr):
                g = r * cpr + ci
                if g + 1 < nch:
                    copies[g + 1].start()
                copies[g].wait()
                buf = bufs[g % 2]

                def body(i, carry, buf=buf):
                    s_c, q_c, mn_c, mx_c = carry
                    s_n, q_n, mn_n, mx_n = [], [], [], []
                    off = i * (_L * _G)
                    for k in range(_G):
                        v = buf[pl.ds(off + k * _L, _L)]
                        s_n.append(s_c[k] + v)
                        q_n.append(q_c[k] + v * v)
                        mn_n.append(jnp.minimum(mn_c[k], v))
                        mx_n.append(jnp.maximum(mx_c[k], v))
                    return (tuple(s_n), tuple(q_n), tuple(mn_n), tuple(mx_n))

                carry = lax.fori_loop(
                    0, iters, body,
                    (tuple(ss), tuple(qq), tuple(mn), tuple(mx)))
                ss, qq, mn, mx = [list(t) for t in carry]

            stat_v[pl.ds(r * 4 * _L + 0 * _L, _L)] = (
                (ss[0] + ss[1]) + (ss[2] + ss[3]))
            stat_v[pl.ds(r * 4 * _L + 1 * _L, _L)] = (
                (qq[0] + qq[1]) + (qq[2] + qq[3]))
            stat_v[pl.ds(r * 4 * _L + 2 * _L, _L)] = jnp.minimum(
                jnp.minimum(mn[0], mn[1]), jnp.minimum(mn[2], mn[3]))
            stat_v[pl.ds(r * 4 * _L + 3 * _L, _L)] = jnp.maximum(
                jnp.maximum(mx[0], mx[1]), jnp.maximum(mx[2], mx[3]))

        pltpu.sync_copy(
            stat_v, out_hbm.at[pl.ds(wid * (rpw * 4 * _L), rpw * 4 * _L)])

    return moments_kernel


def _finish_kernel(n_elems, m_ref, w1t_ref, b1_ref, w2t_ref, b2_ref, o_ref):
    m = m_ref[...]
    n = jnp.float32(n_elems)
    s = jnp.sum(m[:, 0:16], axis=1, keepdims=True)
    q = jnp.sum(m[:, 16:32], axis=1, keepdims=True)
    amin = jnp.min(m[:, 32:48], axis=1, keepdims=True)
    amax = jnp.max(m[:, 48:64], axis=1, keepdims=True)
    mean = s / n
    var = (q - s * s / n) / (n - 1.0)
    std = jnp.sqrt(jnp.maximum(var, 0.0))

    w1t = w1t_ref[...]
    h = (mean * w1t[0:1, :] + std * w1t[1:2, :]
         + amin * w1t[2:3, :] + amax * w1t[3:4, :] + b1_ref[...])
    h = jnp.maximum(h, 0.0)
    logits = jnp.dot(h, w2t_ref[...],
                     preferred_element_type=jnp.float32) + b2_ref[...]

    bsz, k = logits.shape
    col = lax.broadcasted_iota(jnp.int32, (bsz, k), 1)
    m1 = jnp.max(logits, axis=1, keepdims=True)
    i1 = jnp.min(jnp.where(logits == m1, col, k), axis=1, keepdims=True)
    l2 = jnp.where(col == i1, -jnp.inf, logits)
    m2 = jnp.max(l2, axis=1, keepdims=True)
    i2 = jnp.min(jnp.where(l2 == m2, col, k), axis=1, keepdims=True)
    e2 = jnp.exp(m2 - m1)
    denom = 1.0 + e2
    p1 = 1.0 / denom
    p2 = e2 / denom
    o_ref[...] = jnp.where(col == i1, p1,
                           jnp.where(col == i2, p2, jnp.float32(0.0)))


def kernel(u_state, W1, b1, W2, b2):
    batch = u_state.shape[0]
    seq = u_state.shape[1]
    nch = u_state.shape[2]
    row = seq * nch
    chunk = 32768
    if seq % 128 == 0:
        u_flat = (u_state.reshape(batch, seq // 128, 128, nch)
                  .transpose(0, 1, 3, 2).reshape(-1))
    else:
        u_flat = u_state.reshape(-1)

    moments = _make_moments_kernel(batch, row, chunk)(u_flat)
    moments = moments.reshape(batch, 64)

    num_prims = W2.shape[0]
    out = pl.pallas_call(
        functools.partial(_finish_kernel, row),
        out_shape=jax.ShapeDtypeStruct((batch, num_prims), jnp.float32),
    )(moments, W1.T, b1.reshape(1, -1), W2.T, b2.reshape(1, -1))
    return out


# speedup vs baseline: 81.5478x; 1.0001x over previous
import functools

import jax
import jax.numpy as jnp
from jax import lax
from jax.experimental import pallas as pl
from jax.experimental.pallas import tpu as pltpu
from jax.experimental.pallas import tpu_sc as plsc

_NC = 2
_NS = 16
_NW = _NC * _NS
_L = 16
_G = 4


@functools.lru_cache(maxsize=None)
def _make_moments_kernel(batch: int, row: int, chunk: int):
    assert batch % _NW == 0
    rpw = batch // _NW
    assert row % chunk == 0
    cpr = row // chunk
    nch = rpw * cpr
    assert chunk % (_L * _G) == 0
    iters = chunk // (_L * _G)

    mesh = plsc.VectorSubcoreMesh(core_axis_name="c", subcore_axis_name="s")

    @functools.partial(
        pl.kernel,
        out_type=jax.ShapeDtypeStruct((batch * 4 * _L,), jnp.float32),
        mesh=mesh,
        scratch_types=[
            pltpu.VMEM((chunk,), jnp.float32),
            pltpu.VMEM((chunk,), jnp.float32),
            pltpu.VMEM((rpw * 4 * _L,), jnp.float32),
            pltpu.SemaphoreType.DMA,
            pltpu.SemaphoreType.DMA,
        ],
    )
    def moments_kernel(u_hbm, out_hbm, buf0, buf1, stat_v, sem0, sem1):
        cid = lax.axis_index("c")
        sid = lax.axis_index("s")
        wid = sid * _NC + cid
        base = wid * (rpw * row)
        bufs = (buf0, buf1)
        sems = (sem0, sem1)

        def make_copy(g):
            return pltpu.make_async_copy(
                u_hbm.at[pl.ds(base + g * chunk, chunk)], bufs[g % 2],
                sems[g % 2])

        copies = [make_copy(g) for g in range(nch)]
        copies[0].start()

        for r in range(rpw):
            zero = jnp.zeros((_L,), jnp.float32)
            ss = [zero] * _G
            qq = [zero] * _G
            mn = [jnp.full((_L,), jnp.inf, jnp.float32)] * _G
            mx = [jnp.full((_L,), -jnp.inf, jnp.float32)] * _G

            for ci in range(cpr):
                g = r * cpr + ci
                if g + 1 < nch:
                    copies[g + 1].start()
                copies[g].wait()
                buf = bufs[g % 2]

                @plsc.parallel_loop(0, chunk, step=_L * _G, unroll=4,
                                    carry=(tuple(ss), tuple(qq),
                                           tuple(mn), tuple(mx)))
                def body(off, carry, buf=buf):
                    s_c, q_c, mn_c, mx_c = carry
                    s_n, q_n, mn_n, mx_n = [], [], [], []
                    for k in range(_G):
                        v = buf[pl.ds(off + k * _L, _L)]
                        s_n.append(s_c[k] + v)
                        q_n.append(q_c[k] + v * v)
                        mn_n.append(jnp.minimum(mn_c[k], v))
                        mx_n.append(jnp.maximum(mx_c[k], v))
                    return (tuple(s_n), tuple(q_n), tuple(mn_n), tuple(mx_n))

                ss, qq, mn, mx = [list(t) for t in body]

            stat_v[pl.ds(r * 4 * _L + 0 * _L, _L)] = (
                (ss[0] + ss[1]) + (ss[2] + ss[3]))
            stat_v[pl.ds(r * 4 * _L + 1 * _L, _L)] = (
                (qq[0] + qq[1]) + (qq[2] + qq[3]))
            stat_v[pl.ds(r * 4 * _L + 2 * _L, _L)] = jnp.minimum(
                jnp.minimum(mn[0], mn[1]), jnp.minimum(mn[2], mn[3]))
            stat_v[pl.ds(r * 4 * _L + 3 * _L, _L)] = jnp.maximum(
                jnp.maximum(mx[0], mx[1]), jnp.maximum(mx[2], mx[3]))

        pltpu.sync_copy(
            stat_v, out_hbm.at[pl.ds(wid * (rpw * 4 * _L), rpw * 4 * _L)])

    return moments_kernel


def _finish_kernel(n_elems, m_ref, w1t_ref, b1_ref, w2t_ref, b2_ref, o_ref):
    m = m_ref[...]
    n = jnp.float32(n_elems)
    s = jnp.sum(m[:, 0:16], axis=1, keepdims=True)
    q = jnp.sum(m[:, 16:32], axis=1, keepdims=True)
    amin = jnp.min(m[:, 32:48], axis=1, keepdims=True)
    amax = jnp.max(m[:, 48:64], axis=1, keepdims=True)
    mean = s / n
    var = (q - s * s / n) / (n - 1.0)
    std = jnp.sqrt(jnp.maximum(var, 0.0))

    w1t = w1t_ref[...]
    h = (mean * w1t[0:1, :] + std * w1t[1:2, :]
         + amin * w1t[2:3, :] + amax * w1t[3:4, :] + b1_ref[...])
    h = jnp.maximum(h, 0.0)
    logits = jnp.dot(h, w2t_ref[...],
                     preferred_element_type=jnp.float32) + b2_ref[...]

    bsz, k = logits.shape
    col = lax.broadcasted_iota(jnp.int32, (bsz, k), 1)
    m1 = jnp.max(logits, axis=1, keepdims=True)
    i1 = jnp.min(jnp.where(logits == m1, col, k), axis=1, keepdims=True)
    l2 = jnp.where(col == i1, -jnp.inf, logits)
    m2 = jnp.max(l2, axis=1, keepdims=True)
    i2 = jnp.min(jnp.where(l2 == m2, col, k), axis=1, keepdims=True)
    e2 = jnp.exp(m2 - m1)
    denom = 1.0 + e2
    p1 = 1.0 / denom
    p2 = e2 / denom
    o_ref[...] = jnp.where(col == i1, p1,
                           jnp.where(col == i2, p2, jnp.float32(0.0)))


def kernel(u_state, W1, b1, W2, b2):
    batch = u_state.shape[0]
    seq = u_state.shape[1]
    nch = u_state.shape[2]
    row = seq * nch
    chunk = 32768
    if seq % 128 == 0:
        u_flat = (u_state.reshape(batch, seq // 128, 128, nch)
                  .transpose(0, 1, 3, 2).reshape(-1))
    else:
        u_flat = u_state.reshape(-1)

    moments = _make_moments_kernel(batch, row, chunk)(u_flat)
    moments = moments.reshape(batch, 64)

    num_prims = W2.shape[0]
    out = pl.pallas_call(
        functools.partial(_finish_kernel, row),
        out_shape=jax.ShapeDtypeStruct((batch, num_prims), jnp.float32),
    )(moments, W1.T, b1.reshape(1, -1), W2.T, b2.reshape(1, -1))
    return out


# speedup vs baseline: 82.2538x; 1.0087x over previous
import functools

import jax
import jax.numpy as jnp
from jax import lax
from jax.experimental import pallas as pl
from jax.experimental.pallas import tpu as pltpu
from jax.experimental.pallas import tpu_sc as plsc

_NC = 2
_NS = 16
_NW = _NC * _NS
_L = 16
_G = 4


@functools.lru_cache(maxsize=None)
def _make_moments_kernel(batch: int, row: int, chunk: int):
    assert batch % _NW == 0
    rpw = batch // _NW
    assert row % chunk == 0
    cpr = row // chunk
    nch = rpw * cpr
    assert chunk % (_L * _G) == 0
    iters = chunk // (_L * _G)

    mesh = plsc.VectorSubcoreMesh(core_axis_name="c", subcore_axis_name="s")

    @functools.partial(
        pl.kernel,
        out_type=jax.ShapeDtypeStruct((batch * 4 * _L,), jnp.float32),
        mesh=mesh,
        scratch_types=[
            pltpu.VMEM((chunk,), jnp.float32),
            pltpu.VMEM((chunk,), jnp.float32),
            pltpu.VMEM((chunk,), jnp.float32),
            pltpu.VMEM((chunk,), jnp.float32),
            pltpu.VMEM((rpw * 4 * _L,), jnp.float32),
            pltpu.SemaphoreType.DMA,
            pltpu.SemaphoreType.DMA,
            pltpu.SemaphoreType.DMA,
            pltpu.SemaphoreType.DMA,
        ],
    )
    def moments_kernel(u_hbm, out_hbm, buf0, buf1, buf2, buf3, stat_v,
                       sem0, sem1, sem2, sem3):
        cid = lax.axis_index("c")
        sid = lax.axis_index("s")
        wid = sid * _NC + cid
        base = wid * (rpw * row)
        nbuf = 4
        bufs = (buf0, buf1, buf2, buf3)
        sems = (sem0, sem1, sem2, sem3)

        def make_copy(g):
            return pltpu.make_async_copy(
                u_hbm.at[pl.ds(base + g * chunk, chunk)], bufs[g % nbuf],
                sems[g % nbuf])

        copies = [make_copy(g) for g in range(nch)]
        for g in range(min(nbuf - 1, nch)):
            copies[g].start()

        for r in range(rpw):
            zero = jnp.zeros((_L,), jnp.float32)
            ss = [zero] * _G
            qq = [zero] * _G
            mn = [jnp.full((_L,), jnp.inf, jnp.float32)] * _G
            mx = [jnp.full((_L,), -jnp.inf, jnp.float32)] * _G

            for ci in range(cpr):
                g = r * cpr + ci
                if g + nbuf - 1 < nch:
                    copies[g + nbuf - 1].start()
                copies[g].wait()
                buf = bufs[g % nbuf]

                @plsc.parallel_loop(0, chunk, step=_L * _G, unroll=4,
                                    carry=(tuple(ss), tuple(qq),
                                           tuple(mn), tuple(mx)))
                def body(off, carry, buf=buf):
                    s_c, q_c, mn_c, mx_c = carry
                    s_n, q_n, mn_n, mx_n = [], [], [], []
                    for k in range(_G):
                        v = buf[pl.ds(off + k * _L, _L)]
                        s_n.append(s_c[k] + v)
                        q_n.append(q_c[k] + v * v)
                        mn_n.append(jnp.minimum(mn_c[k], v))
                        mx_n.append(jnp.maximum(mx_c[k], v))
                    return (tuple(s_n), tuple(q_n), tuple(mn_n), tuple(mx_n))

                ss, qq, mn, mx = [list(t) for t in body]

            stat_v[pl.ds(r * 4 * _L + 0 * _L, _L)] = (
                (ss[0] + ss[1]) + (ss[2] + ss[3]))
            stat_v[pl.ds(r * 4 * _L + 1 * _L, _L)] = (
                (qq[0] + qq[1]) + (qq[2] + qq[3]))
            stat_v[pl.ds(r * 4 * _L + 2 * _L, _L)] = jnp.minimum(
                jnp.minimum(mn[0], mn[1]), jnp.minimum(mn[2], mn[3]))
            stat_v[pl.ds(r * 4 * _L + 3 * _L, _L)] = jnp.maximum(
                jnp.maximum(mx[0], mx[1]), jnp.maximum(mx[2], mx[3]))

        pltpu.sync_copy(
            stat_v, out_hbm.at[pl.ds(wid * (rpw * 4 * _L), rpw * 4 * _L)])

    return moments_kernel


def _finish_kernel(n_elems, m_ref, w1t_ref, b1_ref, w2t_ref, b2_ref, o_ref):
    m = m_ref[...]
    n = jnp.float32(n_elems)
    s = jnp.sum(m[:, 0:16], axis=1, keepdims=True)
    q = jnp.sum(m[:, 16:32], axis=1, keepdims=True)
    amin = jnp.min(m[:, 32:48], axis=1, keepdims=True)
    amax = jnp.max(m[:, 48:64], axis=1, keepdims=True)
    mean = s / n
    var = (q - s * s / n) / (n - 1.0)
    std = jnp.sqrt(jnp.maximum(var, 0.0))

    w1t = w1t_ref[...]
    h = (mean * w1t[0:1, :] + std * w1t[1:2, :]
         + amin * w1t[2:3, :] + amax * w1t[3:4, :] + b1_ref[...])
    h = jnp.maximum(h, 0.0)
    logits = jnp.dot(h, w2t_ref[...],
                     preferred_element_type=jnp.float32) + b2_ref[...]

    bsz, k = logits.shape
    col = lax.broadcasted_iota(jnp.int32, (bsz, k), 1)
    m1 = jnp.max(logits, axis=1, keepdims=True)
    i1 = jnp.min(jnp.where(logits == m1, col, k), axis=1, keepdims=True)
    l2 = jnp.where(col == i1, -jnp.inf, logits)
    m2 = jnp.max(l2, axis=1, keepdims=True)
    i2 = jnp.min(jnp.where(l2 == m2, col, k), axis=1, keepdims=True)
    e2 = jnp.exp(m2 - m1)
    denom = 1.0 + e2
    p1 = 1.0 / denom
    p2 = e2 / denom
    o_ref[...] = jnp.where(col == i1, p1,
                           jnp.where(col == i2, p2, jnp.float32(0.0)))


def kernel(u_state, W1, b1, W2, b2):
    batch = u_state.shape[0]
    seq = u_state.shape[1]
    nch = u_state.shape[2]
    row = seq * nch
    chunk = 16384
    if seq % 128 == 0:
        u_flat = (u_state.reshape(batch, seq // 128, 128, nch)
                  .transpose(0, 1, 3, 2).reshape(-1))
    else:
        u_flat = u_state.reshape(-1)

    moments = _make_moments_kernel(batch, row, chunk)(u_flat)
    moments = moments.reshape(batch, 64)

    num_prims = W2.shape[0]
    out = pl.pallas_call(
        functools.partial(_finish_kernel, row),
        out_shape=jax.ShapeDtypeStruct((batch, num_prims), jnp.float32),
    )(moments, W1.T, b1.reshape(1, -1), W2.T, b2.reshape(1, -1))
    return out


# speedup vs baseline: 95.2961x; 1.1586x over previous
import functools

import jax
import jax.numpy as jnp
from jax import lax
from jax.experimental import pallas as pl
from jax.experimental.pallas import tpu as pltpu
from jax.experimental.pallas import tpu_sc as plsc

_NC = 2
_NS = 16
_NW = _NC * _NS
_L = 16
_G = 4


def _bcast(x):
    return jnp.broadcast_to(x, (_L,))


@functools.lru_cache(maxsize=None)
def _make_router_kernel(batch: int, row: int, chunk: int, hidden: int,
                        nprim: int):
    assert batch % _NW == 0
    rpw = batch // _NW
    assert row % chunk == 0
    cpr = row // chunk
    nch = rpw * cpr
    assert chunk % (_L * _G) == 0
    assert hidden % _L == 0 and nprim == _L

    mesh = plsc.VectorSubcoreMesh(core_axis_name="c", subcore_axis_name="s")

    @functools.partial(
        pl.kernel,
        out_type=jax.ShapeDtypeStruct((batch * nprim,), jnp.float32),
        mesh=mesh,
        scratch_types=[
            pltpu.VMEM((chunk,), jnp.float32),
            pltpu.VMEM((chunk,), jnp.float32),
            pltpu.VMEM((4 * hidden,), jnp.float32),
            pltpu.VMEM((hidden,), jnp.float32),
            pltpu.VMEM((hidden * nprim,), jnp.float32),
            pltpu.VMEM((nprim,), jnp.float32),
            pltpu.VMEM((rpw * nprim,), jnp.float32),
            pltpu.SemaphoreType.DMA,
            pltpu.SemaphoreType.DMA,
            pltpu.SemaphoreType.DMA,
        ],
        compiler_params=pltpu.CompilerParams(needs_layout_passes=False),
    )
    def router_kernel(u_hbm, w1t_hbm, b1_hbm, w2t_hbm, b2_hbm, out_hbm,
                      buf0, buf1, w1t_v, b1_v, w2t_v, b2_v, o_v,
                      sem0, sem1, semw):
        cid = lax.axis_index("c")
        sid = lax.axis_index("s")
        wid = sid * _NC + cid
        base = wid * (rpw * row)
        bufs = (buf0, buf1)
        sems = (sem0, sem1)

        def make_copy(g):
            return pltpu.make_async_copy(
                u_hbm.at[pl.ds(base + g * chunk, chunk)], bufs[g % 2],
                sems[g % 2])

        copies = [make_copy(g) for g in range(nch)]
        copies[0].start()

        wcopies = [
            pltpu.make_async_copy(w1t_hbm, w1t_v, semw),
            pltpu.make_async_copy(b1_hbm, b1_v, semw),
            pltpu.make_async_copy(w2t_hbm, w2t_v, semw),
            pltpu.make_async_copy(b2_hbm, b2_v, semw),
        ]
        for wc in wcopies:
            wc.start()
        for wc in wcopies:
            wc.wait()

        iota = lax.iota(jnp.int32, _L)
        inv_n = 1.0 / float(row)
        inv_nm1 = 1.0 / float(row - 1)

        for r in range(rpw):
            zero = jnp.zeros((_L,), jnp.float32)
            ss = [zero] * _G
            qq = [zero] * _G
            mn = [jnp.full((_L,), jnp.inf, jnp.float32)] * _G
            mx = [jnp.full((_L,), -jnp.inf, jnp.float32)] * _G

            for ci in range(cpr):
                g = r * cpr + ci
                if g + 1 < nch:
                    copies[g + 1].start()
                copies[g].wait()
                buf = bufs[g % 2]

                @plsc.parallel_loop(0, chunk, step=_L * _G, unroll=4,
                                    carry=(tuple(ss), tuple(qq),
                                           tuple(mn), tuple(mx)))
                def body(off, carry, buf=buf):
                    s_c, q_c, mn_c, mx_c = carry
                    s_n, q_n, mn_n, mx_n = [], [], [], []
                    for k in range(_G):
                        v = buf[pl.ds(off + k * _L, _L)]
                        s_n.append(s_c[k] + v)
                        q_n.append(q_c[k] + v * v)
                        mn_n.append(jnp.minimum(mn_c[k], v))
                        mx_n.append(jnp.maximum(mx_c[k], v))
                    return (tuple(s_n), tuple(q_n), tuple(mn_n), tuple(mx_n))

                ss, qq, mn, mx = [list(t) for t in body]

            s_vec = (ss[0] + ss[1]) + (ss[2] + ss[3])
            q_vec = (qq[0] + qq[1]) + (qq[2] + qq[3])
            mn_vec = jnp.minimum(jnp.minimum(mn[0], mn[1]),
                                 jnp.minimum(mn[2], mn[3]))
            mx_vec = jnp.maximum(jnp.maximum(mx[0], mx[1]),
                                 jnp.maximum(mx[2], mx[3]))

            s_b = _bcast(jnp.sum(s_vec))
            q_b = _bcast(jnp.sum(q_vec))
            meanv = s_b * inv_n
            var_v = (q_b - s_b * s_b * inv_n) * inv_nm1

            vv = jnp.maximum(var_v, 1e-38)
            y = plsc.bitcast(vv, jnp.int32)
            rr = plsc.bitcast(jnp.int32(0x5F3759DF) - (y >> 1), jnp.float32)
            for _ in range(5):
                rr = rr * (1.5 - 0.5 * vv * rr * rr)
            stdv = vv * rr

            mnv = _bcast(jnp.min(mn_vec))
            mxv = _bcast(jnp.max(mx_vec))

            hs = []
            for jj in range(hidden // _L):
                acc = b1_v[pl.ds(jj * _L, _L)]
                acc = acc + meanv * w1t_v[pl.ds(0 * hidden + jj * _L, _L)]
                acc = acc + stdv * w1t_v[pl.ds(1 * hidden + jj * _L, _L)]
                acc = acc + mnv * w1t_v[pl.ds(2 * hidden + jj * _L, _L)]
                acc = acc + mxv * w1t_v[pl.ds(3 * hidden + jj * _L, _L)]
                hs.append(jnp.maximum(acc, 0.0))

            lg = b2_v[pl.ds(0, _L)]
            for j in range(hidden):
                lg = lg + hs[j // _L][j % _L] * w2t_v[pl.ds(j * _L, _L)]

            m1v = _bcast(jnp.max(lg))
            i1 = jnp.min(jnp.where(lg == m1v, iota, _L))
            l2 = jnp.where(iota == i1, jnp.float32(-3.0e38), lg)
            m2v = _bcast(jnp.max(l2))
            i2 = jnp.min(jnp.where(l2 == m2v, iota, _L))
            e2 = jnp.exp(m2v - m1v)
            p1 = 1.0 / (1.0 + e2)
            p2 = e2 * p1
            o_v[pl.ds(r * _L, _L)] = jnp.where(
                iota == i1, p1, jnp.where(iota == i2, p2, jnp.float32(0.0)))

        pltpu.sync_copy(
            o_v, out_hbm.at[pl.ds(wid * (rpw * nprim), rpw * nprim)])

    return router_kernel


def kernel(u_state, W1, b1, W2, b2):
    batch = u_state.shape[0]
    seq = u_state.shape[1]
    nch = u_state.shape[2]
    row = seq * nch
    chunk = 32768
    if seq % 128 == 0:
        u_flat = (u_state.reshape(batch, seq // 128, 128, nch)
                  .transpose(0, 1, 3, 2).reshape(-1))
    else:
        u_flat = u_state.reshape(-1)

    hidden = W1.shape[0]
    nprim = W2.shape[0]
    out = _make_router_kernel(batch, row, chunk, hidden, nprim)(
        u_flat, W1.T.reshape(-1), b1, W2.T.reshape(-1), b2)
    return out.reshape(batch, nprim)
